# Initial kernel scaffold; baseline (speedup 1.0000x reference)
import jax
import jax.numpy as jnp
from jax.experimental import pallas as pl


def kernel(x, edge_index, edge_values):
    raise NotImplementedError("write your pallas kernel here")


# speedup vs baseline: 3.0808x; 3.0808x over previous
import functools

import jax
import jax.numpy as jnp
from jax import lax
from jax.experimental import pallas as pl
from jax.experimental.pallas import tpu as pltpu
from jax.experimental.pallas import tpu_sc as plsc

N = 10000
E = 320000
D = 128
DH = D // 2
NC = 2
NS = 16
EPT = E // NS
K = 80
NCHUNK = EPT // K
RPT = 624
ZR = 208
TAIL = N - NS * RPT
LANES = 16

_mesh = plsc.VectorSubcoreMesh(
    core_axis_name="c", subcore_axis_name="s", num_cores=NC, num_subcores=NS
)


@functools.partial(
    pl.kernel,
    out_type=jax.ShapeDtypeStruct((NC, N, DH), jnp.float32),
    mesh=_mesh,
    scratch_types=[
        pltpu.VMEM((NCHUNK, K), jnp.int32),
        pltpu.VMEM((NCHUNK, K), jnp.int32),
        pltpu.VMEM((NCHUNK, K), jnp.float32),
        pltpu.VMEM((K, DH), jnp.float32),
        pltpu.VMEM((ZR, DH), jnp.float32),
        pltpu.VMEM_SHARED((N, DH), jnp.float32),
    ],
    compiler_params=pltpu.CompilerParams(use_tc_tiling_on_sc=False),
)
def _spmm_sc(xs_hbm, dst_hbm, src_hbm, val_hbm, out_hbm,
             dst_v, src_v, val_v, rows_v, zbuf, acc):
    cid = lax.axis_index("c")
    sid = lax.axis_index("s")

    zero16 = jnp.zeros((LANES,), jnp.float32)

    def zero_row(i, carry):
        for c in range(DH // LANES):
            zbuf[i, pl.ds(c * LANES, LANES)] = zero16
        return carry

    lax.fori_loop(0, ZR, zero_row, 0)
    for t in range(RPT // ZR):
        pltpu.sync_copy(zbuf, acc.at[pl.ds(sid * RPT + t * ZR, ZR)])

    @pl.when(sid == NS - 1)
    def _zero_tail():
        pltpu.sync_copy(zbuf.at[pl.ds(0, TAIL)], acc.at[pl.ds(NS * RPT, TAIL)])

    plsc.subcore_barrier()

    pltpu.sync_copy(dst_hbm.at[sid], dst_v)
    pltpu.sync_copy(src_hbm.at[sid], src_v)
    pltpu.sync_copy(val_hbm.at[sid], val_v)

    def chunk(j, carry):
        pltpu.sync_copy(xs_hbm.at[cid].at[src_v.at[j]], rows_v)

        def scale_group(g, c2):
            v16 = val_v[j, pl.ds(g * LANES, LANES)]
            for jj in range(LANES):
                bv = v16.at[jnp.full((LANES,), jj, jnp.int32)].get(
                    mode="promise_in_bounds")
                e = g * LANES + jj
                for c in range(DH // LANES):
                    sl = pl.ds(c * LANES, LANES)
                    rows_v[e, sl] = rows_v[e, sl] * bv
            return c2

        lax.fori_loop(0, K // LANES, scale_group, 0)
        pltpu.sync_copy(rows_v, acc.at[dst_v.at[j]], add=True)
        return carry

    lax.fori_loop(0, NCHUNK, chunk, 0)

    plsc.subcore_barrier()
    for t in range(RPT // ZR):
        base = sid * RPT + t * ZR
        pltpu.sync_copy(acc.at[pl.ds(base, ZR)],
                        out_hbm.at[cid, pl.ds(base, ZR)])

    @pl.when(sid == NS - 1)
    def _write_tail():
        pltpu.sync_copy(acc.at[pl.ds(NS * RPT, TAIL)],
                        out_hbm.at[cid, pl.ds(NS * RPT, TAIL)])


_BR = 1000


def _combine_body(x_ref, p_ref, o_ref):
    f = jnp.concatenate([p_ref[0], p_ref[1]], axis=-1)
    o_ref[...] = x_ref[...] + jnp.maximum(f, 0.0)


def kernel(x, edge_index, edge_values):
    ei = edge_index.astype(jnp.int32)
    dst3 = ei[0].reshape(NS, NCHUNK, K)
    src3 = ei[1].reshape(NS, NCHUNK, K)
    val3 = edge_values.reshape(NS, NCHUNK, K)
    xs = jnp.stack([x[:, :DH], x[:, DH:]])
    partial = _spmm_sc(xs, dst3, src3, val3)
    return pl.pallas_call(
        _combine_body,
        out_shape=jax.ShapeDtypeStruct((N, D), jnp.float32),
        grid=(N // _BR,),
        in_specs=[
            pl.BlockSpec((_BR, D), lambda i: (i, 0)),
            pl.BlockSpec((NC, _BR, DH), lambda i: (0, i, 0)),
        ],
        out_specs=pl.BlockSpec((_BR, D), lambda i: (i, 0)),
    )(x, partial)
